# baseline (speedup 1.0000x reference)
import functools

import jax
import jax.numpy as jnp
from jax import lax
from jax.experimental import pallas as pl
from jax.experimental.pallas import tpu as pltpu
from jax.experimental.pallas import tpu_sc as plsc

SEQLEN = 65536
TOKENSIZE = 256
WINDOW = 4096

N = WINDOW * TOKENSIZE
FLAT = SEQLEN * TOKENSIZE
NC, NS = 2, 16
NW = NC * NS
CHUNK = N // NW
L = 16


@functools.partial(
    pl.kernel,
    mesh=plsc.VectorSubcoreMesh(core_axis_name="c", subcore_axis_name="s"),
    out_type=jax.ShapeDtypeStruct((N,), jnp.float32),
    scratch_types=[
        pltpu.VMEM((CHUNK,), jnp.int32),
        pltpu.VMEM((CHUNK,), jnp.int32),
        pltpu.VMEM((CHUNK,), jnp.float32),
        pltpu.VMEM((CHUNK,), jnp.float32),
        pltpu.SemaphoreType.DMA,
        pltpu.SemaphoreType.DMA,
    ],
    compiler_params=pltpu.CompilerParams(needs_layout_passes=False),
)
def _sc_gather(table_hbm, widx_hbm, half_hbm, out_hbm, widx_v, half_v, word_v,
               val_v, sem, sem2):
    wid = lax.axis_index("s") * NC + lax.axis_index("c")
    base = wid * CHUNK
    halves = pltpu.make_async_copy(half_hbm.at[pl.ds(base, CHUNK)], half_v, sem2)
    halves.start()
    pltpu.sync_copy(widx_hbm.at[pl.ds(base, CHUNK)], widx_v)
    pltpu.async_copy(table_hbm.at[widx_v], word_v, sem).wait()
    halves.wait()

    def body(i, _):
        s = pl.ds(i * L, L)
        w = plsc.bitcast(word_v[s], jnp.int32)
        sh = half_v[s] << 4
        val_v[s] = plsc.bitcast((w >> sh) << 16, jnp.float32)
        return ()

    lax.fori_loop(0, CHUNK // L, body, (), unroll=8)
    pltpu.sync_copy(val_v, out_hbm.at[pl.ds(base, CHUNK)])


def _pack_table(ddx):
    pairs = ddx.reshape(SEQLEN, 128, 2).astype(jnp.bfloat16)
    return jax.lax.bitcast_convert_type(pairs, jnp.float32).reshape(FLAT // 2)


def kernel(ddx, mdx, p, perm):
    table = _pack_table(ddx)
    e = jax.lax.slice(perm, (0, 0), (WINDOW, TOKENSIZE)).reshape(N)
    widx = e >> 1
    half = e & 1
    wdx = _sc_gather(table, widx, half)
    ddx_out = jnp.stack([ddx[:WINDOW], wdx.reshape(WINDOW, TOKENSIZE)])
    return (ddx_out, mdx[:WINDOW], p[:WINDOW])


# speedup vs baseline: 3.7079x; 3.7079x over previous
import functools

import jax
import jax.numpy as jnp
from jax import lax
from jax.experimental import pallas as pl
from jax.experimental.pallas import tpu as pltpu
from jax.experimental.pallas import tpu_sc as plsc

SEQLEN = 65536
TOKENSIZE = 256
WINDOW = 4096

N = WINDOW * TOKENSIZE
FLAT = SEQLEN * TOKENSIZE
NC, NS = 2, 16
NW = NC * NS
CHUNK = N // NW
SUB = CHUNK // 2


@functools.partial(
    pl.kernel,
    mesh=plsc.VectorSubcoreMesh(core_axis_name="c", subcore_axis_name="s"),
    out_type=jax.ShapeDtypeStruct((N,), jnp.float32),
    scratch_types=[
        pltpu.VMEM((SUB,), jnp.int32),
        pltpu.VMEM((SUB,), jnp.int32),
        pltpu.VMEM((SUB,), jnp.float32),
        pltpu.VMEM((SUB,), jnp.float32),
        pltpu.SemaphoreType.DMA,
        pltpu.SemaphoreType.DMA,
        pltpu.SemaphoreType.DMA,
    ],
)
def _sc_gather(table_hbm, idx_hbm, out_hbm, idx_a, idx_b, val_a, val_b,
               gsem, isem, osem):
    wid = lax.axis_index("s") * NC + lax.axis_index("c")
    base = wid * CHUNK
    pltpu.sync_copy(idx_hbm.at[pl.ds(base, SUB)], idx_a)
    ga = pltpu.make_async_copy(table_hbm.at[idx_a], val_a, gsem)
    ga.start()
    ld_b = pltpu.make_async_copy(idx_hbm.at[pl.ds(base + SUB, SUB)], idx_b, isem)
    ld_b.start()
    ld_b.wait()
    ga.wait()
    gb = pltpu.make_async_copy(table_hbm.at[idx_b], val_b, gsem)
    gb.start()
    wr_a = pltpu.make_async_copy(val_a, out_hbm.at[pl.ds(base, SUB)], osem)
    wr_a.start()
    gb.wait()
    wr_a.wait()
    pltpu.sync_copy(val_b, out_hbm.at[pl.ds(base + SUB, SUB)])


_RL_ROWS = 4096


def _relayout_body(x_ref, o_ref):
    o_ref[...] = x_ref[...].reshape(2 * _RL_ROWS, 128)


def _tc_relayout(ddx):
    return pl.pallas_call(
        _relayout_body,
        grid=(SEQLEN // _RL_ROWS,),
        in_specs=[pl.BlockSpec((_RL_ROWS, TOKENSIZE), lambda i: (i, 0))],
        out_specs=pl.BlockSpec((2 * _RL_ROWS, 128), lambda i: (i, 0)),
        out_shape=jax.ShapeDtypeStruct((FLAT // 128, 128), jnp.float32),
    )(ddx)


def kernel(ddx, mdx, p, perm):
    idx = jax.lax.slice(perm, (0, 0), (WINDOW, TOKENSIZE)).reshape(N)
    ddx_q, idx = jax.lax.optimization_barrier((ddx, idx))
    table = _tc_relayout(ddx_q).reshape(FLAT)
    wdx = _sc_gather(table, idx)
    ddx_out = jnp.stack([ddx[:WINDOW], wdx.reshape(WINDOW, TOKENSIZE)])
    return (ddx_out, mdx[:WINDOW], p[:WINDOW])


# speedup vs baseline: 3.7492x; 1.0111x over previous
import functools

import jax
import jax.numpy as jnp
from jax import lax
from jax.experimental import pallas as pl
from jax.experimental.pallas import tpu as pltpu
from jax.experimental.pallas import tpu_sc as plsc

SEQLEN = 65536
TOKENSIZE = 256
WINDOW = 4096

N = WINDOW * TOKENSIZE
FLAT = SEQLEN * TOKENSIZE
NC, NS = 2, 16
NW = NC * NS
CHUNK = N // NW
SUB = CHUNK // 2


@functools.partial(
    pl.kernel,
    mesh=plsc.VectorSubcoreMesh(core_axis_name="c", subcore_axis_name="s"),
    out_type=jax.ShapeDtypeStruct((N,), jnp.float32),
    scratch_types=[
        pltpu.VMEM((SUB,), jnp.int32),
        pltpu.VMEM((SUB,), jnp.int32),
        pltpu.VMEM((SUB,), jnp.float32),
        pltpu.VMEM((SUB,), jnp.float32),
        pltpu.SemaphoreType.DMA,
        pltpu.SemaphoreType.DMA,
        pltpu.SemaphoreType.DMA,
    ],
)
def _sc_gather(table_hbm, idx_hbm, out_hbm, idx_a, idx_b, val_a, val_b,
               gsem, isem, osem):
    wid = lax.axis_index("s") * NC + lax.axis_index("c")
    base = wid * CHUNK
    pltpu.sync_copy(idx_hbm.at[pl.ds(base, SUB)], idx_a)
    ga = pltpu.make_async_copy(table_hbm.at[idx_a], val_a, gsem)
    ga.start()
    ld_b = pltpu.make_async_copy(idx_hbm.at[pl.ds(base + SUB, SUB)], idx_b, isem)
    ld_b.start()
    ld_b.wait()
    ga.wait()
    gb = pltpu.make_async_copy(table_hbm.at[idx_b], val_b, gsem)
    gb.start()
    wr_a = pltpu.make_async_copy(val_a, out_hbm.at[pl.ds(base, SUB)], osem)
    wr_a.start()
    gb.wait()
    wr_a.wait()
    pltpu.sync_copy(val_b, out_hbm.at[pl.ds(base + SUB, SUB)])


_RL_ROWS = 8192


def _relayout_body(x_ref, o_ref):
    o_ref[...] = x_ref[...].reshape(2 * _RL_ROWS, 128)


def _tc_relayout(ddx):
    return pl.pallas_call(
        _relayout_body,
        grid=(SEQLEN // _RL_ROWS,),
        in_specs=[pl.BlockSpec((_RL_ROWS, TOKENSIZE), lambda i: (i, 0))],
        out_specs=pl.BlockSpec((2 * _RL_ROWS, 128), lambda i: (i, 0)),
        out_shape=jax.ShapeDtypeStruct((FLAT // 128, 128), jnp.float32),
    )(ddx)


def kernel(ddx, mdx, p, perm):
    idx = jax.lax.slice(perm, (0, 0), (WINDOW, TOKENSIZE)).reshape(N)
    ddx_q, idx = jax.lax.optimization_barrier((ddx, idx))
    table = _tc_relayout(ddx_q).reshape(FLAT)
    wdx = _sc_gather(table, idx)
    ddx_out = jnp.stack([ddx[:WINDOW], wdx.reshape(WINDOW, TOKENSIZE)])
    return (ddx_out, mdx[:WINDOW], p[:WINDOW])


# speedup vs baseline: 3.9111x; 1.0432x over previous
import functools

import jax
import jax.numpy as jnp
from jax import lax
from jax.experimental import pallas as pl
from jax.experimental.pallas import tpu as pltpu
from jax.experimental.pallas import tpu_sc as plsc

SEQLEN = 65536
TOKENSIZE = 256
WINDOW = 4096

N = WINDOW * TOKENSIZE
FLAT = SEQLEN * TOKENSIZE
NC, NS = 2, 16
NW = NC * NS
CHUNK = N // NW
SUB = CHUNK // 2


@functools.partial(
    pl.kernel,
    mesh=plsc.VectorSubcoreMesh(core_axis_name="c", subcore_axis_name="s"),
    out_type=jax.ShapeDtypeStruct((N,), jnp.float32),
    scratch_types=[
        pltpu.VMEM((SUB,), jnp.int32),
        pltpu.VMEM((SUB,), jnp.int32),
        pltpu.VMEM((SUB,), jnp.float32),
        pltpu.VMEM((SUB,), jnp.float32),
        pltpu.SemaphoreType.DMA,
        pltpu.SemaphoreType.DMA,
        pltpu.SemaphoreType.DMA,
    ],
)
def _sc_gather(table_hbm, idx_hbm, out_hbm, idx_a, idx_b, val_a, val_b,
               gsem, isem, osem):
    wid = lax.axis_index("s") * NC + lax.axis_index("c")
    base = wid * CHUNK
    pltpu.sync_copy(idx_hbm.at[pl.ds(base, SUB)], idx_a)
    ga = pltpu.make_async_copy(table_hbm.at[idx_a], val_a, gsem)
    ga.start()
    ld_b = pltpu.make_async_copy(idx_hbm.at[pl.ds(base + SUB, SUB)], idx_b, isem)
    ld_b.start()
    ld_b.wait()
    ga.wait()
    gb = pltpu.make_async_copy(table_hbm.at[idx_b], val_b, gsem)
    gb.start()
    wr_a = pltpu.make_async_copy(val_a, out_hbm.at[pl.ds(base, SUB)], osem)
    wr_a.start()
    gb.wait()
    wr_a.wait()
    pltpu.sync_copy(val_b, out_hbm.at[pl.ds(base + SUB, SUB)])


_RL_ROWS = 8192


def _relayout_body(x_ref, o_ref):
    o_ref[...] = x_ref[...].reshape(2 * _RL_ROWS, 128)


def _tc_relayout(ddx):
    return pl.pallas_call(
        _relayout_body,
        grid=(SEQLEN // _RL_ROWS,),
        in_specs=[pl.BlockSpec((_RL_ROWS, TOKENSIZE), lambda i: (i, 0))],
        out_specs=pl.BlockSpec((2 * _RL_ROWS, 128), lambda i: (i, 0)),
        out_shape=jax.ShapeDtypeStruct((FLAT // 128, 128), jnp.float32),
    )(ddx)


_AS_ROWS = 512


def _assemble_body(win_ref, phys_ref, o_ref):
    o_ref[0] = win_ref[...]
    x = phys_ref[...].reshape(_AS_ROWS // 8, 16, 128)
    o_ref[1] = jnp.concatenate([x[:, :8], x[:, 8:]], axis=2).reshape(
        _AS_ROWS, TOKENSIZE
    )


def _tc_assemble(ddx, wdx_phys):
    return pl.pallas_call(
        _assemble_body,
        grid=(WINDOW // _AS_ROWS,),
        in_specs=[
            pl.BlockSpec((_AS_ROWS, TOKENSIZE), lambda i: (i, 0)),
            pl.BlockSpec((2 * _AS_ROWS, 128), lambda i: (i, 0)),
        ],
        out_specs=pl.BlockSpec((2, _AS_ROWS, TOKENSIZE), lambda i: (0, i, 0)),
        out_shape=jax.ShapeDtypeStruct((2, WINDOW, TOKENSIZE), jnp.float32),
    )(ddx, wdx_phys)


def kernel(ddx, mdx, p, perm):
    idx = (
        jax.lax.slice(perm, (0, 0), (WINDOW, TOKENSIZE))
        .reshape(WINDOW // 8, 8, 2, 128)
        .transpose(0, 2, 1, 3)
        .reshape(N)
    )
    ddx_q, idx = jax.lax.optimization_barrier((ddx, idx))
    table = _tc_relayout(ddx_q).reshape(FLAT)
    wdx_phys = _sc_gather(table, idx).reshape(N // 128, 128)
    ddx_out = _tc_assemble(ddx, wdx_phys)
    return (ddx_out, mdx[:WINDOW], p[:WINDOW])


# speedup vs baseline: 3.9861x; 1.0192x over previous
import functools

import jax
import jax.numpy as jnp
from jax import lax
from jax.experimental import pallas as pl
from jax.experimental.pallas import tpu as pltpu
from jax.experimental.pallas import tpu_sc as plsc

SEQLEN = 65536
TOKENSIZE = 256
WINDOW = 4096

N = WINDOW * TOKENSIZE
FLAT = SEQLEN * TOKENSIZE
NC, NS = 2, 16
NW = NC * NS
CHUNK = N // NW
SUB = CHUNK // 2


@functools.partial(
    pl.kernel,
    mesh=plsc.VectorSubcoreMesh(core_axis_name="c", subcore_axis_name="s"),
    out_type=jax.ShapeDtypeStruct((N,), jnp.float32),
    scratch_types=[
        pltpu.VMEM((SUB,), jnp.int32),
        pltpu.VMEM((SUB,), jnp.int32),
        pltpu.VMEM((SUB,), jnp.float32),
        pltpu.VMEM((SUB,), jnp.float32),
        pltpu.SemaphoreType.DMA,
        pltpu.SemaphoreType.DMA,
        pltpu.SemaphoreType.DMA,
    ],
)
def _sc_gather(table_hbm, idx_hbm, out_hbm, idx_a, idx_b, val_a, val_b,
               gsem, isem, osem):
    wid = lax.axis_index("s") * NC + lax.axis_index("c")
    base = wid * CHUNK
    pltpu.sync_copy(idx_hbm.at[pl.ds(base, SUB)], idx_a)
    ga = pltpu.make_async_copy(table_hbm.at[idx_a], val_a, gsem)
    ga.start()
    ld_b = pltpu.make_async_copy(idx_hbm.at[pl.ds(base + SUB, SUB)], idx_b, isem)
    ld_b.start()
    ld_b.wait()
    ga.wait()
    gb = pltpu.make_async_copy(table_hbm.at[idx_b], val_b, gsem)
    gb.start()
    wr_a = pltpu.make_async_copy(val_a, out_hbm.at[pl.ds(base, SUB)], osem)
    wr_a.start()
    gb.wait()
    wr_a.wait()
    pltpu.sync_copy(val_b, out_hbm.at[pl.ds(base + SUB, SUB)])


_RL_ROWS = 8192


_IP_ROWS = WINDOW // (SEQLEN // _RL_ROWS)


def _relayout_body(x_ref, perm_ref, o_ref, idx_ref):
    o_ref[...] = x_ref[...].reshape(2 * _RL_ROWS, 128)
    y = perm_ref[...].reshape(_IP_ROWS // 8, 8, 2, 128)
    idx_ref[...] = y.transpose(0, 2, 1, 3).reshape(_IP_ROWS * TOKENSIZE)


def _tc_relayout(ddx, perm):
    return pl.pallas_call(
        _relayout_body,
        grid=(SEQLEN // _RL_ROWS,),
        in_specs=[
            pl.BlockSpec((_RL_ROWS, TOKENSIZE), lambda i: (i, 0)),
            pl.BlockSpec((_IP_ROWS, TOKENSIZE), lambda i: (i, 0)),
        ],
        out_specs=[
            pl.BlockSpec((2 * _RL_ROWS, 128), lambda i: (i, 0)),
            pl.BlockSpec((_IP_ROWS * TOKENSIZE,), lambda i: (i,)),
        ],
        out_shape=[
            jax.ShapeDtypeStruct((FLAT // 128, 128), jnp.float32),
            jax.ShapeDtypeStruct((N,), jnp.int32),
        ],
    )(ddx, perm)


_AS_ROWS = 512


def _assemble_body(win_ref, phys_ref, o_ref):
    o_ref[0] = win_ref[...]
    x = phys_ref[...].reshape(_AS_ROWS // 8, 16, 128)
    o_ref[1] = jnp.concatenate([x[:, :8], x[:, 8:]], axis=2).reshape(
        _AS_ROWS, TOKENSIZE
    )


def _tc_assemble(ddx, wdx_phys):
    return pl.pallas_call(
        _assemble_body,
        grid=(WINDOW // _AS_ROWS,),
        in_specs=[
            pl.BlockSpec((_AS_ROWS, TOKENSIZE), lambda i: (i, 0)),
            pl.BlockSpec((2 * _AS_ROWS, 128), lambda i: (i, 0)),
        ],
        out_specs=pl.BlockSpec((2, _AS_ROWS, TOKENSIZE), lambda i: (0, i, 0)),
        out_shape=jax.ShapeDtypeStruct((2, WINDOW, TOKENSIZE), jnp.float32),
    )(ddx, wdx_phys)


def kernel(ddx, mdx, p, perm):
    table2d, idx = _tc_relayout(ddx, perm)
    wdx_phys = _sc_gather(table2d.reshape(FLAT), idx).reshape(N // 128, 128)
    ddx_out = _tc_assemble(ddx, wdx_phys)
    return (ddx_out, mdx[:WINDOW], p[:WINDOW])


# speedup vs baseline: 4.0444x; 1.0146x over previous
import functools

import jax
import jax.numpy as jnp
from jax import lax
from jax.experimental import pallas as pl
from jax.experimental.pallas import tpu as pltpu
from jax.experimental.pallas import tpu_sc as plsc

SEQLEN = 65536
TOKENSIZE = 256
WINDOW = 4096

N = WINDOW * TOKENSIZE
FLAT = SEQLEN * TOKENSIZE
NC, NS = 2, 16
NW = NC * NS
CHUNK = N // NW
SUB = CHUNK // 2


@functools.partial(
    pl.kernel,
    mesh=plsc.VectorSubcoreMesh(core_axis_name="c", subcore_axis_name="s"),
    out_type=jax.ShapeDtypeStruct((N,), jnp.float32),
    scratch_types=[
        pltpu.VMEM((SUB,), jnp.int32),
        pltpu.VMEM((SUB,), jnp.int32),
        pltpu.VMEM((SUB,), jnp.float32),
        pltpu.VMEM((SUB,), jnp.float32),
        pltpu.SemaphoreType.DMA,
        pltpu.SemaphoreType.DMA,
        pltpu.SemaphoreType.DMA,
    ],
)
def _sc_gather(table_hbm, idx_hbm, out_hbm, idx_a, idx_b, val_a, val_b,
               gsem, isem, osem):
    wid = lax.axis_index("s") * NC + lax.axis_index("c")
    base = wid * CHUNK
    pltpu.sync_copy(idx_hbm.at[pl.ds(base, SUB)], idx_a)
    ga = pltpu.make_async_copy(table_hbm.at[idx_a], val_a, gsem)
    ga.start()
    ld_b = pltpu.make_async_copy(idx_hbm.at[pl.ds(base + SUB, SUB)], idx_b, isem)
    ld_b.start()
    ld_b.wait()
    ga.wait()
    gb = pltpu.make_async_copy(table_hbm.at[idx_b], val_b, gsem)
    gb.start()
    wr_a = pltpu.make_async_copy(val_a, out_hbm.at[pl.ds(base, SUB)], osem)
    wr_a.start()
    gb.wait()
    wr_a.wait()
    pltpu.sync_copy(val_b, out_hbm.at[pl.ds(base + SUB, SUB)])


_RL_ROWS = 8192


_IP_ROWS = WINDOW // (SEQLEN // _RL_ROWS)


def _relayout_body(x_ref, perm_ref, o_ref, idx_ref):
    o_ref[...] = x_ref[...].reshape(2 * _RL_ROWS, 128)
    y = perm_ref[...].reshape(_IP_ROWS // 8, 8, 2, 128)
    idx_ref[...] = y.transpose(0, 2, 1, 3).reshape(_IP_ROWS * TOKENSIZE)


def _tc_relayout(ddx, perm):
    return pl.pallas_call(
        _relayout_body,
        grid=(SEQLEN // _RL_ROWS,),
        in_specs=[
            pl.BlockSpec((_RL_ROWS, TOKENSIZE), lambda i: (i, 0)),
            pl.BlockSpec((_IP_ROWS, TOKENSIZE), lambda i: (i, 0)),
        ],
        out_specs=[
            pl.BlockSpec((2 * _RL_ROWS, 128), lambda i: (i, 0)),
            pl.BlockSpec((_IP_ROWS * TOKENSIZE,), lambda i: (i,)),
        ],
        out_shape=[
            jax.ShapeDtypeStruct((FLAT // 128, 128), jnp.float32),
            jax.ShapeDtypeStruct((N,), jnp.int32),
        ],
    )(ddx, perm)


_AS_ROWS = 1024


def _assemble_body(win_ref, phys_ref, o_ref):
    o_ref[0] = win_ref[...]
    x = phys_ref[...].reshape(_AS_ROWS // 8, 16, 128)
    o_ref[1] = jnp.concatenate([x[:, :8], x[:, 8:]], axis=2).reshape(
        _AS_ROWS, TOKENSIZE
    )


def _tc_assemble(ddx, wdx_phys):
    return pl.pallas_call(
        _assemble_body,
        grid=(WINDOW // _AS_ROWS,),
        in_specs=[
            pl.BlockSpec((_AS_ROWS, TOKENSIZE), lambda i: (i, 0)),
            pl.BlockSpec((2 * _AS_ROWS, 128), lambda i: (i, 0)),
        ],
        out_specs=pl.BlockSpec((2, _AS_ROWS, TOKENSIZE), lambda i: (0, i, 0)),
        out_shape=jax.ShapeDtypeStruct((2, WINDOW, TOKENSIZE), jnp.float32),
    )(ddx, wdx_phys)


def kernel(ddx, mdx, p, perm):
    table2d, idx = _tc_relayout(ddx, perm)
    wdx_phys = _sc_gather(table2d.reshape(FLAT), idx).reshape(N // 128, 128)
    ddx_out = _tc_assemble(ddx, wdx_phys)
    return (ddx_out, mdx[:WINDOW], p[:WINDOW])
